# baseline (speedup 1.0000x reference)
import jax
import jax.numpy as jnp
from jax import lax
from jax.experimental import pallas as pl
from jax.experimental.pallas import tpu as pltpu
from jax.experimental.pallas import tpu_sc as plsc

E, R, S = 100, 100000, 16384
NC, NS, L = 2, 16, 16
NW = NC * NS
H = 2
RH = R // H
CH = 4096
NCH = S // CH
TASKS = E * 2 * H
NT = (TASKS + NW - 1) // NW


def _body(a_hbm, b_hbm, srt_hbm, da_hbm, db_hbm, outa_hbm, outb_hbm,
          row0, row1, idx0, idx1, val0, val1,
          sin0, sin1, sout0, sout1, sidx0, sidx1, sval0, sval1):
    wid = lax.axis_index("s") * NC + lax.axis_index("c")
    rows, idxb, valb = (row0, row1), (idx0, idx1), (val0, val1)
    sins, souts = (sin0, sin1), (sout0, sout1)
    sidxs, svals = (sidx0, sidx1), (sval0, sval1)

    def parts(t):
        tid = (t * NW + wid) % TASKS
        sub = tid % (2 * H)
        return tid // (2 * H), sub // H, sub % H

    def row_off(e, h):
        return pl.multiple_of(e * R + h * RH, 8)

    def start_in(t):
        e, arr, h = parts(t)
        off = row_off(e, h)

        @pl.when(arr == 0)
        def _():
            pltpu.async_copy(a_hbm.at[pl.ds(off, RH)], rows[t % 2],
                             sins[t % 2])

        @pl.when(arr == 1)
        def _():
            pltpu.async_copy(b_hbm.at[pl.ds(off, RH)], rows[t % 2],
                             sins[t % 2])

    def wait_in(t):
        pltpu.make_async_copy(a_hbm.at[pl.ds(0, RH)], rows[t % 2],
                              sins[t % 2]).wait()

    def start_out(t):
        e, arr, h = parts(t)
        off = row_off(e, h)

        @pl.when(arr == 0)
        def _():
            pltpu.async_copy(rows[t % 2], outa_hbm.at[pl.ds(off, RH)],
                             souts[t % 2])

        @pl.when(arr == 1)
        def _():
            pltpu.async_copy(rows[t % 2], outb_hbm.at[pl.ds(off, RH)],
                             souts[t % 2])

    def wait_out(t):
        pltpu.make_async_copy(rows[t % 2], outa_hbm.at[pl.ds(0, RH)],
                              souts[t % 2]).wait()

    def prefetch_chunk(t, c):
        e, arr, _ = parts(t)
        ioff = pl.multiple_of(e * S + c * CH, 8)
        pltpu.async_copy(srt_hbm.at[pl.ds(ioff, CH)], idxb[c % 2],
                         sidxs[c % 2])

        @pl.when(arr == 0)
        def _():
            pltpu.async_copy(da_hbm.at[pl.ds(c * CH, CH)], valb[c % 2],
                             svals[c % 2])

        @pl.when(arr == 1)
        def _():
            pltpu.async_copy(db_hbm.at[pl.ds(c * CH, CH)], valb[c % 2],
                             svals[c % 2])

    def wait_chunk(c):
        pltpu.make_async_copy(srt_hbm.at[pl.ds(0, CH)], idxb[c % 2],
                              sidxs[c % 2]).wait()
        pltpu.make_async_copy(da_hbm.at[pl.ds(0, CH)], valb[c % 2],
                              svals[c % 2]).wait()

    prefetch_chunk(0, 0)
    start_in(0)
    for t in range(NT):
        if t >= 1:
            wait_out(t - 1)
        if t + 1 < NT:
            start_in(t + 1)
        wait_in(t)
        _, _, h = parts(t)
        base = h * RH
        rowbuf = rows[t % 2]
        for c in range(NCH):
            wait_chunk(c)
            if c + 1 < NCH:
                prefetch_chunk(t, c + 1)
            elif t + 1 < NT:
                prefetch_chunk(t + 1, 0)
            idxc, valc = idxb[c % 2], valb[c % 2]

            def _inner(i, carry, idxc=idxc, valc=valc, base=base,
                       rowbuf=rowbuf):
                idx = idxc[pl.ds(i * L, L)]
                v = valc[pl.ds(i * L, L)]
                mask = (idx >= base) & (idx < base + RH)
                plsc.addupdate_scatter(rowbuf, [idx - base], v, mask=mask)
                return carry

            lax.fori_loop(0, CH // L, _inner, 0, unroll=4)
        start_out(t)
    wait_out(NT - 1)


_scatter_update = pl.kernel(
    _body,
    out_type=[jax.ShapeDtypeStruct((E * R,), jnp.float32),
              jax.ShapeDtypeStruct((E * R,), jnp.float32)],
    mesh=plsc.VectorSubcoreMesh(core_axis_name="c", subcore_axis_name="s",
                                num_cores=NC, num_subcores=NS),
    scratch_types=[pltpu.VMEM((RH,), jnp.float32),
                   pltpu.VMEM((RH,), jnp.float32),
                   pltpu.VMEM((CH,), jnp.int32),
                   pltpu.VMEM((CH,), jnp.int32),
                   pltpu.VMEM((CH,), jnp.float32),
                   pltpu.VMEM((CH,), jnp.float32)]
                  + [pltpu.SemaphoreType.DMA] * 8,
    compiler_params=pltpu.CompilerParams(needs_layout_passes=False),
)


@jax.jit
def kernel(a, b, samples_regions, da, db):
    srt = samples_regions.T.reshape(E * S)
    na, nb = _scatter_update(a.reshape(E * R), b.reshape(E * R), srt, da, db)
    return na.reshape(E, R), nb.reshape(E, R)


# speedup vs baseline: 2.4941x; 2.4941x over previous
import jax
import jax.numpy as jnp
from jax import lax
from jax.experimental import pallas as pl
from jax.experimental.pallas import tpu as pltpu
from jax.experimental.pallas import tpu_sc as plsc

E, R, S = 100, 100000, 16384
NC, NS, L = 2, 16, 16
NW = NC * NS
VCHUNK = 8192


def _body(a_hbm, b_hbm, srt_hbm, da_hbm, db_hbm, outa_hbm, outb_hbm,
          row_v, idx_v, val_v):
    wid = lax.axis_index("s") * NC + lax.axis_index("c")
    for k in range((E + NW - 1) // NW):
        e = wid + k * NW

        @pl.when(e < E)
        def _process():
            pltpu.sync_copy(srt_hbm.at[e], idx_v)
            for src, dst, vals in ((a_hbm, outa_hbm, da_hbm),
                                   (b_hbm, outb_hbm, db_hbm)):
                pltpu.sync_copy(src.at[e], row_v)
                for c in range(S // VCHUNK):
                    pltpu.sync_copy(vals.at[pl.ds(c * VCHUNK, VCHUNK)], val_v)

                    def _inner(i, carry, c=c):
                        idx = idx_v[pl.ds(c * VCHUNK + i * L, L)]
                        v = val_v[pl.ds(i * L, L)]
                        plsc.addupdate_scatter(row_v, [idx], v)
                        return carry

                    lax.fori_loop(0, VCHUNK // L, _inner, 0, unroll=8)
                pltpu.sync_copy(row_v, dst.at[e])


_scatter_update = pl.kernel(
    _body,
    out_type=[jax.ShapeDtypeStruct((E, R), jnp.float32),
              jax.ShapeDtypeStruct((E, R), jnp.float32)],
    mesh=plsc.VectorSubcoreMesh(core_axis_name="c", subcore_axis_name="s",
                                num_cores=NC, num_subcores=NS),
    scratch_types=[pltpu.VMEM((R,), jnp.float32),
                   pltpu.VMEM((S,), jnp.int32),
                   pltpu.VMEM((VCHUNK,), jnp.float32)],
    compiler_params=pltpu.CompilerParams(needs_layout_passes=False),
)


@jax.jit
def kernel(a, b, samples_regions, da, db):
    srt = samples_regions.T
    return tuple(_scatter_update(a, b, srt, da, db))
